# Initial kernel scaffold; baseline (speedup 1.0000x reference)
import jax
import jax.numpy as jnp
from jax.experimental import pallas as pl


def kernel(keys, values, importance, enc0_W, enc0_b, enc1_W, enc1_b, enc2_W, enc2_b, dec0_W, dec0_b, dec1_W, dec1_b, dec2_W, dec2_b):
    raise NotImplementedError("write your pallas kernel here")


# speedup vs baseline: 7.9194x; 7.9194x over previous
import functools

import jax
import jax.numpy as jnp
from jax.experimental import pallas as pl

HID = 1024
N_TOK = 4 * 4096
K1 = 3276
K2 = K1 + 2097
ROWS = 128
COLS = N_TOK // ROWS
BITS_HI = 0x3F800000

BLK = 512
H1P = 256


def _levels_body(bits_ref, lvl_ref):
    bits = bits_ref[...]

    def step(_, carry):
        lo1, hi1, lo2, hi2 = carry
        mid1 = lo1 + (hi1 - lo1) // 2
        mid2 = lo2 + (hi2 - lo2) // 2
        c1 = jnp.sum((bits > mid1).astype(jnp.int32))
        c2 = jnp.sum((bits > mid2).astype(jnp.int32))
        p1 = c1 < K1
        p2 = c2 < K2
        return (jnp.where(p1, lo1, mid1), jnp.where(p1, mid1, hi1),
                jnp.where(p2, lo2, mid2), jnp.where(p2, mid2, hi2))

    init = (jnp.int32(-1), jnp.int32(BITS_HI), jnp.int32(-1), jnp.int32(BITS_HI))
    _, v1, _, v2 = jax.lax.fori_loop(0, 31, step, init)

    n_gt1 = jnp.sum((bits > v1).astype(jnp.int32))
    n_gt2 = jnp.sum((bits > v2).astype(jnp.int32))

    io_r = jax.lax.broadcasted_iota(jnp.int32, (ROWS, COLS), 0)
    io_c = jax.lax.broadcasted_iota(jnp.int32, (ROWS, COLS), 1)
    s_upper = (io_r < io_c).astype(jnp.float32)
    s_lower = (io_c < io_r).astype(jnp.float32)

    def excl_prefix(eq_f):
        in_row = jnp.dot(eq_f, s_upper, preferred_element_type=jnp.float32)
        row_off = jnp.sum(
            jnp.dot(s_lower, eq_f, preferred_element_type=jnp.float32),
            axis=1, keepdims=True)
        return (in_row + row_off).astype(jnp.int32)

    eq1 = bits == v1
    eq2 = bits == v2
    rank1 = n_gt1 + excl_prefix(eq1.astype(jnp.float32))
    rank2 = n_gt2 + excl_prefix(eq2.astype(jnp.float32))

    lvl = jnp.where(bits > v1, 0, jnp.where(bits > v2, 1, 2)).astype(jnp.int32)
    lvl = jnp.where(eq1, jnp.where(rank1 < K1, 0, jnp.where(rank1 < K2, 1, 2)), lvl)
    lvl = jnp.where(eq2 & (v1 != v2), jnp.where(rank2 < K2, 1, 2), lvl)
    lvl_ref[...] = lvl


def _compute_levels(importance):
    scaled = importance / (jnp.max(importance) + 1e-05)
    bits = jax.lax.bitcast_convert_type(scaled, jnp.int32).reshape(ROWS, COLS)
    lvl = pl.pallas_call(
        _levels_body,
        out_shape=jax.ShapeDtypeStruct((ROWS, COLS), jnp.int32),
    )(bits)
    return lvl.reshape(N_TOK, 1)


def _mlp_body(lvl_ref, k_ref, v_ref, w0_ref, b0_ref, w1_ref, b1_ref,
              d1w_ref, d1b_ref, d0w_ref, d0b_ref, ok_ref, ov_ref):
    lvl = lvl_ref[...]
    w0 = w0_ref[...]
    w1 = w1_ref[...]
    d1w = d1w_ref[...]
    d0w = d0w_ref[...]
    b0 = b0_ref[...]
    b1 = b1_ref[...]
    d1b = d1b_ref[...]
    d0b = d0b_ref[...]

    def transform(x):
        h0 = jnp.maximum(jnp.dot(x, w0, preferred_element_type=jnp.float32) + b0, 0.0)
        h1 = jnp.maximum(jnp.dot(h0, w1, preferred_element_type=jnp.float32) + b1, 0.0)
        g1 = jnp.maximum(jnp.dot(h1, d1w, preferred_element_type=jnp.float32) + d1b, 0.0)
        z = jnp.where(lvl == 1, h0, g1)
        y = jnp.maximum(jnp.dot(z, d0w, preferred_element_type=jnp.float32) + d0b, 0.0)
        return jnp.where(lvl == 0, x, y)

    ok_ref[...] = transform(k_ref[...])
    ov_ref[...] = transform(v_ref[...])


def kernel(keys, values, importance, enc0_W, enc0_b, enc1_W, enc1_b,
           enc2_W, enc2_b, dec0_W, dec0_b, dec1_W, dec1_b, dec2_W, dec2_b):
    B, S, H = keys.shape
    lvl = _compute_levels(importance.reshape(-1))

    kf = keys.reshape(N_TOK, H)
    vf = values.reshape(N_TOK, H)

    pad1 = H1P - enc1_W.shape[0]
    w0 = enc0_W.T
    b0 = enc0_b.reshape(1, -1)
    w1 = jnp.pad(enc1_W, ((0, pad1), (0, 0))).T
    b1 = jnp.pad(enc1_b, (0, pad1)).reshape(1, -1)
    d1w = jnp.pad(dec1_W, ((0, 0), (0, pad1))).T
    d1b = dec1_b.reshape(1, -1)
    d0w = dec0_W.T
    d0b = dec0_b.reshape(1, -1)

    grid = (N_TOK // BLK,)
    row_spec = pl.BlockSpec((BLK, H), lambda i: (i, 0))
    lvl_spec = pl.BlockSpec((BLK, 1), lambda i: (i, 0))

    def full(a):
        return pl.BlockSpec(a.shape, lambda i: (0,) * a.ndim)

    ok, ov = pl.pallas_call(
        _mlp_body,
        grid=grid,
        in_specs=[lvl_spec, row_spec, row_spec,
                  full(w0), full(b0), full(w1), full(b1),
                  full(d1w), full(d1b), full(d0w), full(d0b)],
        out_specs=[row_spec, row_spec],
        out_shape=[jax.ShapeDtypeStruct((N_TOK, H), jnp.float32),
                   jax.ShapeDtypeStruct((N_TOK, H), jnp.float32)],
    )(lvl, kf, vf, w0, b0, w1, b1, d1w, d1b, d0w, d0b)

    return (ok.reshape(B, S, H), ov.reshape(B, S, H))


# speedup vs baseline: 8.4504x; 1.0671x over previous
import functools

import jax
import jax.numpy as jnp
from jax.experimental import pallas as pl

HID = 1024
N_TOK = 4 * 4096
K1 = 3276
K2 = K1 + 2097
ROWS = 128
COLS = N_TOK // ROWS
BITS_HI = 0x3F800000

BLK = 1024
H1P = 256


def _levels_body(bits_ref, lvl_ref):
    bits = bits_ref[...]

    def step(_, carry):
        lo1, hi1, lo2, hi2 = carry
        mid1 = lo1 + (hi1 - lo1) // 2
        mid2 = lo2 + (hi2 - lo2) // 2
        c1 = jnp.sum((bits > mid1).astype(jnp.int32))
        c2 = jnp.sum((bits > mid2).astype(jnp.int32))
        p1 = c1 < K1
        p2 = c2 < K2
        return (jnp.where(p1, lo1, mid1), jnp.where(p1, mid1, hi1),
                jnp.where(p2, lo2, mid2), jnp.where(p2, mid2, hi2))

    init = (jnp.int32(-1), jnp.int32(BITS_HI), jnp.int32(-1), jnp.int32(BITS_HI))
    _, v1, _, v2 = jax.lax.fori_loop(0, 31, step, init)

    n_gt1 = jnp.sum((bits > v1).astype(jnp.int32))
    n_gt2 = jnp.sum((bits > v2).astype(jnp.int32))

    io_r = jax.lax.broadcasted_iota(jnp.int32, (ROWS, COLS), 0)
    io_c = jax.lax.broadcasted_iota(jnp.int32, (ROWS, COLS), 1)
    s_upper = (io_r < io_c).astype(jnp.float32)
    s_lower = (io_c < io_r).astype(jnp.float32)

    def excl_prefix(eq_f):
        in_row = jnp.dot(eq_f, s_upper, preferred_element_type=jnp.float32)
        row_off = jnp.sum(
            jnp.dot(s_lower, eq_f, preferred_element_type=jnp.float32),
            axis=1, keepdims=True)
        return (in_row + row_off).astype(jnp.int32)

    eq1 = bits == v1
    eq2 = bits == v2
    rank1 = n_gt1 + excl_prefix(eq1.astype(jnp.float32))
    rank2 = n_gt2 + excl_prefix(eq2.astype(jnp.float32))

    lvl = jnp.where(bits > v1, 0, jnp.where(bits > v2, 1, 2)).astype(jnp.int32)
    lvl = jnp.where(eq1, jnp.where(rank1 < K1, 0, jnp.where(rank1 < K2, 1, 2)), lvl)
    lvl = jnp.where(eq2 & (v1 != v2), jnp.where(rank2 < K2, 1, 2), lvl)
    lvl_ref[...] = lvl


def _compute_levels(importance):
    scaled = importance / (jnp.max(importance) + 1e-05)
    bits = jax.lax.bitcast_convert_type(scaled, jnp.int32).reshape(ROWS, COLS)
    lvl = pl.pallas_call(
        _levels_body,
        out_shape=jax.ShapeDtypeStruct((ROWS, COLS), jnp.int32),
    )(bits)
    return lvl.reshape(N_TOK, 1)


def _mlp_body(lvl_ref, k_ref, v_ref, w0_ref, b0_ref, w1_ref, b1_ref,
              d1w_ref, d1b_ref, d0w_ref, d0b_ref, ok_ref, ov_ref):
    lvl = lvl_ref[...]
    is1 = lvl == 1
    is0 = lvl == 0
    w0 = w0_ref[...]
    w1 = w1_ref[...]
    d1w = d1w_ref[...]
    d0w = d0w_ref[...]
    b0 = b0_ref[...]
    b1 = b1_ref[...]
    d1b = d1b_ref[...]
    d0b = d0b_ref[...]

    def transform(x):
        h0 = jnp.maximum(jnp.dot(x, w0, preferred_element_type=jnp.float32) + b0, 0.0)
        h1 = jnp.maximum(jnp.dot(h0, w1, preferred_element_type=jnp.float32) + b1, 0.0)
        g1 = jnp.maximum(jnp.dot(h1, d1w, preferred_element_type=jnp.float32) + d1b, 0.0)
        z = jnp.where(is1, h0, g1)
        y = jnp.maximum(jnp.dot(z, d0w, preferred_element_type=jnp.float32) + d0b, 0.0)
        return jnp.where(is0, x, y)

    ok_ref[...] = transform(k_ref[...])
    ov_ref[...] = transform(v_ref[...])


def kernel(keys, values, importance, enc0_W, enc0_b, enc1_W, enc1_b,
           enc2_W, enc2_b, dec0_W, dec0_b, dec1_W, dec1_b, dec2_W, dec2_b):
    B, S, H = keys.shape
    lvl = _compute_levels(importance.reshape(-1))

    kf = keys.reshape(N_TOK, H)
    vf = values.reshape(N_TOK, H)

    pad1 = H1P - enc1_W.shape[0]
    w0 = enc0_W.T
    b0 = enc0_b.reshape(1, -1)
    w1 = jnp.pad(enc1_W, ((0, pad1), (0, 0))).T
    b1 = jnp.pad(enc1_b, (0, pad1)).reshape(1, -1)
    d1w = jnp.pad(dec1_W, ((0, 0), (0, pad1))).T
    d1b = dec1_b.reshape(1, -1)
    d0w = dec0_W.T
    d0b = dec0_b.reshape(1, -1)

    grid = (N_TOK // BLK,)
    row_spec = pl.BlockSpec((BLK, H), lambda i: (i, 0))
    lvl_spec = pl.BlockSpec((BLK, 1), lambda i: (i, 0))

    def full(a):
        return pl.BlockSpec(a.shape, lambda i: (0,) * a.ndim)

    ok, ov = pl.pallas_call(
        _mlp_body,
        grid=grid,
        in_specs=[lvl_spec, row_spec, row_spec,
                  full(w0), full(b0), full(w1), full(b1),
                  full(d1w), full(d1b), full(d0w), full(d0b)],
        out_specs=[row_spec, row_spec],
        out_shape=[jax.ShapeDtypeStruct((N_TOK, H), jnp.float32),
                   jax.ShapeDtypeStruct((N_TOK, H), jnp.float32)],
    )(lvl, kf, vf, w0, b0, w1, b1, d1w, d1b, d0w, d0b)

    return (ok.reshape(B, S, H), ov.reshape(B, S, H))


# speedup vs baseline: 8.5939x; 1.0170x over previous
import functools

import jax
import jax.numpy as jnp
from jax.experimental import pallas as pl

HID = 1024
N_TOK = 4 * 4096
K1 = 3276
K2 = K1 + 2097
ROWS = 128
COLS = N_TOK // ROWS
BITS_HI = 0x3F800000

BLK = 1024
H1P = 256


def _levels_body(bits_ref, lvl_ref):
    bits = bits_ref[...]

    def step(_, carry):
        lo1, hi1, lo2, hi2 = carry
        mid1 = lo1 + (hi1 - lo1) // 2
        mid2 = lo2 + (hi2 - lo2) // 2
        c1 = jnp.sum((bits > mid1).astype(jnp.int32))
        c2 = jnp.sum((bits > mid2).astype(jnp.int32))
        p1 = c1 < K1
        p2 = c2 < K2
        return (jnp.where(p1, lo1, mid1), jnp.where(p1, mid1, hi1),
                jnp.where(p2, lo2, mid2), jnp.where(p2, mid2, hi2))

    init = (jnp.int32(-1), jnp.int32(BITS_HI), jnp.int32(-1), jnp.int32(BITS_HI))
    _, v1, _, v2 = jax.lax.fori_loop(0, 31, step, init)

    n_gt1 = jnp.sum((bits > v1).astype(jnp.int32))
    n_gt2 = jnp.sum((bits > v2).astype(jnp.int32))

    io_r = jax.lax.broadcasted_iota(jnp.int32, (ROWS, COLS), 0)
    io_c = jax.lax.broadcasted_iota(jnp.int32, (ROWS, COLS), 1)
    s_upper = (io_r < io_c).astype(jnp.float32)
    s_lower = (io_c < io_r).astype(jnp.float32)

    def excl_prefix(eq_f):
        in_row = jnp.dot(eq_f, s_upper, preferred_element_type=jnp.float32)
        row_off = jnp.sum(
            jnp.dot(s_lower, eq_f, preferred_element_type=jnp.float32),
            axis=1, keepdims=True)
        return (in_row + row_off).astype(jnp.int32)

    eq1 = bits == v1
    eq2 = bits == v2
    rank1 = n_gt1 + excl_prefix(eq1.astype(jnp.float32))
    rank2 = n_gt2 + excl_prefix(eq2.astype(jnp.float32))

    lvl = jnp.where(bits > v1, 0, jnp.where(bits > v2, 1, 2)).astype(jnp.int32)
    lvl = jnp.where(eq1, jnp.where(rank1 < K1, 0, jnp.where(rank1 < K2, 1, 2)), lvl)
    lvl = jnp.where(eq2 & (v1 != v2), jnp.where(rank2 < K2, 1, 2), lvl)
    lvl_ref[...] = lvl


def _compute_levels(importance):
    scaled = importance / (jnp.max(importance) + 1e-05)
    bits = jax.lax.bitcast_convert_type(scaled, jnp.int32).reshape(ROWS, COLS)
    lvl = pl.pallas_call(
        _levels_body,
        out_shape=jax.ShapeDtypeStruct((ROWS, COLS), jnp.int32),
    )(bits)
    return lvl.reshape(N_TOK, 1).astype(jnp.int8)


def _mlp_body(lvl_ref, k_ref, v_ref, w0_ref, b0_ref, w1_ref, b1_ref,
              d1w_ref, d1b_ref, d0w_ref, d0b_ref, ok_ref, ov_ref):
    lvl = lvl_ref[...]
    is1 = lvl == 1
    is0 = lvl == 0
    w0 = w0_ref[...]
    w1 = w1_ref[...]
    d1w = d1w_ref[...]
    d0w = d0w_ref[...]
    b0 = b0_ref[...]
    b1 = b1_ref[...]
    d1b = d1b_ref[...]
    d0b = d0b_ref[...]

    def transform(x):
        h0 = jnp.maximum(jnp.dot(x, w0, preferred_element_type=jnp.float32) + b0, 0.0)
        h1 = jnp.maximum(jnp.dot(h0, w1, preferred_element_type=jnp.float32) + b1, 0.0)
        g1 = jnp.maximum(jnp.dot(h1, d1w, preferred_element_type=jnp.float32) + d1b, 0.0)
        z = jnp.where(is1, h0, g1)
        y = jnp.maximum(jnp.dot(z, d0w, preferred_element_type=jnp.float32) + d0b, 0.0)
        return jnp.where(is0, x, y)

    ok_ref[...] = transform(k_ref[...])
    ov_ref[...] = transform(v_ref[...])


def kernel(keys, values, importance, enc0_W, enc0_b, enc1_W, enc1_b,
           enc2_W, enc2_b, dec0_W, dec0_b, dec1_W, dec1_b, dec2_W, dec2_b):
    B, S, H = keys.shape
    lvl = _compute_levels(importance.reshape(-1))

    kf = keys.reshape(N_TOK, H)
    vf = values.reshape(N_TOK, H)

    pad1 = H1P - enc1_W.shape[0]
    w0 = enc0_W.T
    b0 = enc0_b.reshape(1, -1)
    w1 = jnp.pad(enc1_W, ((0, pad1), (0, 0))).T
    b1 = jnp.pad(enc1_b, (0, pad1)).reshape(1, -1)
    d1w = jnp.pad(dec1_W, ((0, 0), (0, pad1))).T
    d1b = dec1_b.reshape(1, -1)
    d0w = dec0_W.T
    d0b = dec0_b.reshape(1, -1)

    grid = (N_TOK // BLK,)
    row_spec = pl.BlockSpec((BLK, H), lambda i: (i, 0))
    lvl_spec = pl.BlockSpec((BLK, 1), lambda i: (i, 0))

    def full(a):
        return pl.BlockSpec(a.shape, lambda i: (0,) * a.ndim)

    ok, ov = pl.pallas_call(
        _mlp_body,
        grid=grid,
        in_specs=[lvl_spec, row_spec, row_spec,
                  full(w0), full(b0), full(w1), full(b1),
                  full(d1w), full(d1b), full(d0w), full(d0b)],
        out_specs=[row_spec, row_spec],
        out_shape=[jax.ShapeDtypeStruct((N_TOK, H), jnp.float32),
                   jax.ShapeDtypeStruct((N_TOK, H), jnp.float32)],
    )(lvl, kf, vf, w0, b0, w1, b1, d1w, d1b, d0w, d0b)

    return (ok.reshape(B, S, H), ov.reshape(B, S, H))
